# baseline (speedup 1.0000x reference)
import functools

import jax
import jax.numpy as jnp
from jax import lax
from jax.experimental import pallas as pl
from jax.experimental.pallas import tpu as pltpu
from jax.experimental.pallas import tpu_sc as plsc

B = 16384
D = 32
NC = 2
NS = 16
NW = NC * NS
BPW = B // NW

_sc_mesh = plsc.VectorSubcoreMesh(core_axis_name="c", subcore_axis_name="s")


@functools.partial(
    pl.kernel,
    mesh=_sc_mesh,
    compiler_params=pltpu.CompilerParams(use_tc_tiling_on_sc=False),
    out_type=[
        jax.ShapeDtypeStruct((D, B), jnp.float32),
        jax.ShapeDtypeStruct((D, B), jnp.float32),
    ],
    scratch_types=[
        pltpu.VMEM((BPW,), jnp.int32),
        pltpu.VMEM((BPW,), jnp.int32),
        pltpu.VMEM((D, BPW), jnp.float32),
        pltpu.VMEM((D, BPW), jnp.float32),
        pltpu.SemaphoreType.DMA,
    ],
)
def _sc_gather(user_hbm, item_hbm, utT_hbm, itT_hbm, ueT_hbm, ieT_hbm,
               uidx_v, iidx_v, urows_v, irows_v, sem):
    wid = lax.axis_index("s") * NC + lax.axis_index("c")
    base = wid * BPW
    pltpu.sync_copy(user_hbm.at[pl.ds(base, BPW)], uidx_v)
    pltpu.sync_copy(item_hbm.at[pl.ds(base, BPW)], iidx_v)
    copies = []
    for d in range(D):
        copies.append(
            pltpu.async_copy(utT_hbm.at[d].at[uidx_v], urows_v.at[d], sem))
        copies.append(
            pltpu.async_copy(itT_hbm.at[d].at[iidx_v], irows_v.at[d], sem))
    for c in copies:
        c.wait()
    pltpu.sync_copy(urows_v, ueT_hbm.at[:, pl.ds(base, BPW)])
    pltpu.sync_copy(irows_v, ieT_hbm.at[:, pl.ds(base, BPW)])


TB = 2048


def _tc_dense_body(ueT_ref, ieT_ref, fmW_ref, w1_ref, b1_ref,
                   w2_ref, b2_ref, w3_ref, bias_ref, out_ref):
    ueT = ueT_ref[...]
    ieT = ieT_ref[...]
    interT = ueT * ieT
    wu = fmW_ref[:, :D]
    wi = fmW_ref[:, D:]
    fm = (jnp.dot(wu, ueT, preferred_element_type=jnp.float32)
          + jnp.dot(wi, ieT, preferred_element_type=jnp.float32))
    h = jnp.maximum(
        jnp.dot(w1_ref[...], interT, preferred_element_type=jnp.float32)
        + b1_ref[...].reshape(-1, 1), 0.0)
    h = jnp.maximum(
        jnp.dot(w2_ref[...], h, preferred_element_type=jnp.float32)
        + b2_ref[...].reshape(-1, 1), 0.0)
    deep = jnp.dot(w3_ref[...], h, preferred_element_type=jnp.float32)
    logit = fm[0, :] + deep[0, :] + bias_ref[0]
    out_ref[...] = 1.0 / (1.0 + jnp.exp(-logit))


def _tc_dense(ueT, ieT, fm_W, W1, b1, W2, b2, W3, bias):
    grid = (B // TB,)
    return pl.pallas_call(
        _tc_dense_body,
        grid=grid,
        in_specs=[
            pl.BlockSpec((D, TB), lambda i: (0, i)),
            pl.BlockSpec((D, TB), lambda i: (0, i)),
            pl.BlockSpec(fm_W.shape, lambda i: (0, 0)),
            pl.BlockSpec(W1.shape, lambda i: (0, 0)),
            pl.BlockSpec(b1.shape, lambda i: (0,)),
            pl.BlockSpec(W2.shape, lambda i: (0, 0)),
            pl.BlockSpec(b2.shape, lambda i: (0,)),
            pl.BlockSpec(W3.shape, lambda i: (0, 0)),
            pl.BlockSpec(bias.shape, lambda i: (0,)),
        ],
        out_specs=pl.BlockSpec((TB,), lambda i: (i,)),
        out_shape=jax.ShapeDtypeStruct((B,), jnp.float32),
    )(ueT, ieT, fm_W, W1, b1, W2, b2, W3, bias)


def kernel(user, item, user_table, item_table, fm_W, fm_b, W1, b1, W2, b2, W3, b3):
    user = user.astype(jnp.int32)
    item = item.astype(jnp.int32)
    utT = user_table.T
    itT = item_table.T
    ueT, ieT = _sc_gather(user, item, utT, itT)
    bias = (fm_b + b3).reshape((1,))
    return _tc_dense(ueT, ieT, fm_W, W1, b1, W2, b2, W3, bias)
